# baseline (speedup 1.0000x reference)
import functools

import jax
import jax.numpy as jnp
from jax import lax
from jax.experimental import pallas as pl
from jax.experimental.pallas import tpu as pltpu
from jax.experimental.pallas import tpu_sc as plsc

_NE = 1_000_000
_D = 64
_C = 100
_B = 16384
_NW = 32
_EB = 4 * _B // _NW
_RB = 2 * _B // _NW
_CHUNK = 128

_PB = 16384
_QB = _PB // 4
_SH = 12
_NSTEP = (_NE + _PB - 1) // _PB
_NP = _NSTEP * _QB


def _prep_body(et, ct, w1, w2, bias, out):
    dn = (((0,), (1,)), ((), ()))
    f = lax.dot_general(et[...].astype(jnp.bfloat16), w1[...], dn,
                        preferred_element_type=jnp.float32)
    f = f + lax.dot_general(ct[...].astype(jnp.bfloat16), w2[...], dn,
                            preferred_element_type=jnp.float32)
    f = jnp.maximum(f + bias[...], 0.0)
    fb = f.astype(jnp.bfloat16)
    h = jnp.concatenate([fb[:_PB // 2], fb[_PB // 2:]], axis=1)
    out[...] = pltpu.bitcast(h, jnp.float32)


_prep = pl.pallas_call(
    _prep_body,
    grid=(_NSTEP,),
    in_specs=[
        pl.BlockSpec((_D, _PB), lambda j: (0, j)),
        pl.BlockSpec((_C, _PB), lambda j: (0, j)),
        pl.BlockSpec((_D, _D), lambda j: (0, 0)),
        pl.BlockSpec((_D, _C), lambda j: (0, 0)),
        pl.BlockSpec((1, _D), lambda j: (0, 0)),
    ],
    out_specs=pl.BlockSpec((_QB, 2 * _D), lambda j: (j, 0)),
    out_shape=jax.ShapeDtypeStruct((_NP, 2 * _D), jnp.float32),
)


def _sc_gather_x_body(eidx, ptab, x_out, idx_v, buf0, buf1, sem0, sem1):
    cid = lax.axis_index("c")
    sid = lax.axis_index("s")
    wid = sid * 2 + cid
    ebase = wid * _EB

    pltpu.sync_copy(eidx.at[pl.ds(ebase, _EB)], idx_v)

    def tstep(k, carry):
        sl = pl.ds(k * 16, 16)
        v = idx_v[sl]
        idx_v[sl] = ((v >> 14) << 12) | ((v & 8191) >> 1)
        return carry

    lax.fori_loop(0, _EB // 16, tstep, 0)

    def desc(j, buf, sem):
        return pltpu.make_async_copy(
            ptab.at[idx_v.at[pl.ds(j * _CHUNK, _CHUNK)]], buf, sem)

    nch = _EB // _CHUNK
    desc(0, buf0, sem0).start()

    def estep(j2, carry):
        c = 2 * j2
        desc(c + 1, buf1, sem1).start()
        desc(c, buf0, sem0).wait()
        pltpu.sync_copy(buf0, x_out.at[pl.ds(ebase + c * _CHUNK, _CHUNK)])

        @pl.when(j2 < nch // 2 - 1)
        def _():
            desc(c + 2, buf0, sem0).start()

        desc(c + 1, buf1, sem1).wait()
        pltpu.sync_copy(buf1, x_out.at[pl.ds(ebase + (c + 1) * _CHUNK, _CHUNK)])
        return carry

    lax.fori_loop(0, nch // 2, estep, 0)


_sc_gather_x = functools.partial(
    pl.kernel,
    out_type=jax.ShapeDtypeStruct((4 * _B, 2 * _D), jnp.float32),
    mesh=plsc.VectorSubcoreMesh(core_axis_name="c", subcore_axis_name="s"),
    compiler_params=pltpu.CompilerParams(use_tc_tiling_on_sc=False),
    scratch_types=[
        pltpu.VMEM((_EB,), jnp.int32),
        pltpu.VMEM((_CHUNK, 2 * _D), jnp.float32),
        pltpu.VMEM((_CHUNK, 2 * _D), jnp.float32),
        pltpu.SemaphoreType.DMA,
        pltpu.SemaphoreType.DMA,
    ],
)(_sc_gather_x_body)


def _sc_gather_rel_body(ridx, rtab, r_out, idx_v, buf0, buf1, sem0, sem1):
    cid = lax.axis_index("c")
    sid = lax.axis_index("s")
    wid = sid * 2 + cid
    rbase = wid * _RB

    pltpu.sync_copy(ridx.at[pl.ds(rbase, _RB)], idx_v)

    def desc(j, buf, sem):
        return pltpu.make_async_copy(
            rtab.at[idx_v.at[pl.ds(j * _CHUNK, _CHUNK)]], buf, sem)

    nch = _RB // _CHUNK
    desc(0, buf0, sem0).start()

    def rstep(j2, carry):
        c = 2 * j2
        desc(c + 1, buf1, sem1).start()
        desc(c, buf0, sem0).wait()
        pltpu.sync_copy(buf0, r_out.at[pl.ds(rbase + c * _CHUNK, _CHUNK)])

        @pl.when(j2 < nch // 2 - 1)
        def _():
            desc(c + 2, buf0, sem0).start()

        desc(c + 1, buf1, sem1).wait()
        pltpu.sync_copy(buf1, r_out.at[pl.ds(rbase + (c + 1) * _CHUNK, _CHUNK)])
        return carry

    lax.fori_loop(0, nch // 2, rstep, 0)


_sc_gather_rel = functools.partial(
    pl.kernel,
    out_type=jax.ShapeDtypeStruct((2 * _B, _D), jnp.float32),
    mesh=plsc.VectorSubcoreMesh(core_axis_name="c", subcore_axis_name="s"),
    compiler_params=pltpu.CompilerParams(use_tc_tiling_on_sc=False),
    scratch_types=[
        pltpu.VMEM((_RB,), jnp.int32),
        pltpu.VMEM((_CHUNK, _D), jnp.float32),
        pltpu.VMEM((_CHUNK, _D), jnp.float32),
        pltpu.SemaphoreType.DMA,
        pltpu.SemaphoreType.DMA,
    ],
)(_sc_gather_rel_body)


_BLK = 1024
_NB = _B // _BLK


def _score_body(xh, xt, xnh, xnt, ih, it, inh, intt, rp, rn, pos_o, neg_o):
    def sel(x_ref, i_ref):
        xb = pltpu.bitcast(x_ref[...], jnp.bfloat16).reshape(_BLK, 2, 2 * _D)
        i = i_ref[...]
        par = (i & 1)[:, None]
        row = jnp.where(par == 0, xb[:, 0, :], xb[:, 1, :])
        half = ((i >> 13) & 1)[:, None]
        return jnp.where(half == 0, row[:, :_D], row[:, _D:]).astype(jnp.float32)

    fh = sel(xh, ih)
    ft = sel(xt, it)
    fnh = sel(xnh, inh)
    fnt = sel(xnt, intt)
    pos_o[...] = jnp.sum(jnp.abs(fh + rp[...] - ft), axis=1)
    neg_o[...] = jnp.sum(jnp.abs(fnh + rn[...] - fnt), axis=1)


def _x_spec(seg):
    return pl.BlockSpec((_BLK, 2 * _D), lambda i, s=seg: (i + s * _NB, 0))


def _i_spec(seg):
    return pl.BlockSpec((_BLK,), lambda i: (i,))


def _r_spec(seg):
    return pl.BlockSpec((_BLK, _D), lambda i, s=seg: (i + s * _NB, 0))


_score = pl.pallas_call(
    _score_body,
    grid=(_NB,),
    in_specs=[
        _x_spec(0), _x_spec(1), _x_spec(2), _x_spec(3),
        _i_spec(0), _i_spec(1), _i_spec(2), _i_spec(3),
        _r_spec(0), _r_spec(1),
    ],
    out_specs=[
        pl.BlockSpec((_BLK,), lambda i: (i,)),
        pl.BlockSpec((_BLK,), lambda i: (i,)),
    ],
    out_shape=[
        jax.ShapeDtypeStruct((_B,), jnp.float32),
        jax.ShapeDtypeStruct((_B,), jnp.float32),
    ],
)


def kernel(pos_h, pos_r, pos_t, neg_h, neg_r, neg_t, entity_table,
           relation_table, commonsense_table, W_fuse, b_fuse):
    eidx = jnp.concatenate([pos_h, pos_t, neg_h, neg_t])
    ridx = jnp.concatenate([pos_r, neg_r])
    ptab = _prep(entity_table.T, commonsense_table.T,
                 W_fuse[:, :_D].astype(jnp.bfloat16),
                 W_fuse[:, _D:].astype(jnp.bfloat16), b_fuse.reshape(1, _D))
    r_rows = _sc_gather_rel(ridx, relation_table)
    x_rows = _sc_gather_x(eidx, ptab)
    pos, neg = _score(
        x_rows, x_rows, x_rows, x_rows,
        pos_h, pos_t, neg_h, neg_t,
        r_rows, r_rows)
    return (pos, neg)


# speedup vs baseline: 1.2334x; 1.2334x over previous
import functools

import jax
import jax.numpy as jnp
from jax import lax
from jax.experimental import pallas as pl
from jax.experimental.pallas import tpu as pltpu
from jax.experimental.pallas import tpu_sc as plsc

_NE = 1_000_000
_D = 64
_C = 100
_B = 16384
_NW = 32
_EB = 4 * _B // _NW
_RB = 2 * _B // _NW
_CHUNK = 128

_PB = 16384
_HPB = _PB // 2
_SH = 13
_NSTEP = (_NE + _PB - 1) // _PB
_NP = _NSTEP * _HPB


def _prep_body(et, ct, w1, w2, bias, out):
    dn = (((0,), (1,)), ((), ()))
    f = lax.dot_general(et[...].astype(jnp.bfloat16), w1[...], dn,
                        preferred_element_type=jnp.float32)
    f = f + lax.dot_general(ct[...].astype(jnp.bfloat16), w2[...], dn,
                            preferred_element_type=jnp.float32)
    f = jnp.maximum(f + bias[...], 0.0)
    out[...] = jnp.concatenate([f[:_HPB], f[_HPB:]], axis=1)


_prep = pl.pallas_call(
    _prep_body,
    grid=(_NSTEP,),
    in_specs=[
        pl.BlockSpec((_D, _PB), lambda j: (0, j)),
        pl.BlockSpec((_C, _PB), lambda j: (0, j)),
        pl.BlockSpec((_D, _D), lambda j: (0, 0)),
        pl.BlockSpec((_D, _C), lambda j: (0, 0)),
        pl.BlockSpec((1, _D), lambda j: (0, 0)),
    ],
    out_specs=pl.BlockSpec((_HPB, 2 * _D), lambda j: (j, 0)),
    out_shape=jax.ShapeDtypeStruct((_NP, 2 * _D), jnp.float32),
)


def _sc_gather_x_body(eidx, ptab, x_out, idx_v, buf0, buf1, sem0, sem1):
    cid = lax.axis_index("c")
    sid = lax.axis_index("s")
    wid = sid * 2 + cid
    ebase = wid * _EB

    pltpu.sync_copy(eidx.at[pl.ds(ebase, _EB)], idx_v)

    def tstep(k, carry):
        sl = pl.ds(k * 16, 16)
        v = idx_v[sl]
        idx_v[sl] = ((v >> (_SH + 1)) << _SH) | (v & (_HPB - 1))
        return carry

    lax.fori_loop(0, _EB // 16, tstep, 0)

    def desc(j, buf, sem):
        return pltpu.make_async_copy(
            ptab.at[idx_v.at[pl.ds(j * _CHUNK, _CHUNK)]], buf, sem)

    nch = _EB // _CHUNK
    desc(0, buf0, sem0).start()

    def estep(j2, carry):
        c = 2 * j2
        desc(c + 1, buf1, sem1).start()
        desc(c, buf0, sem0).wait()
        pltpu.sync_copy(buf0, x_out.at[pl.ds(ebase + c * _CHUNK, _CHUNK)])

        @pl.when(j2 < nch // 2 - 1)
        def _():
            desc(c + 2, buf0, sem0).start()

        desc(c + 1, buf1, sem1).wait()
        pltpu.sync_copy(buf1, x_out.at[pl.ds(ebase + (c + 1) * _CHUNK, _CHUNK)])
        return carry

    lax.fori_loop(0, nch // 2, estep, 0)


_sc_gather_x = functools.partial(
    pl.kernel,
    out_type=jax.ShapeDtypeStruct((4 * _B, 2 * _D), jnp.float32),
    mesh=plsc.VectorSubcoreMesh(core_axis_name="c", subcore_axis_name="s"),
    compiler_params=pltpu.CompilerParams(use_tc_tiling_on_sc=False),
    scratch_types=[
        pltpu.VMEM((_EB,), jnp.int32),
        pltpu.VMEM((_CHUNK, 2 * _D), jnp.float32),
        pltpu.VMEM((_CHUNK, 2 * _D), jnp.float32),
        pltpu.SemaphoreType.DMA,
        pltpu.SemaphoreType.DMA,
    ],
)(_sc_gather_x_body)


def _sc_gather_rel_body(ridx, rtab, r_out, idx_v, buf0, buf1, sem0, sem1):
    cid = lax.axis_index("c")
    sid = lax.axis_index("s")
    wid = sid * 2 + cid
    rbase = wid * _RB

    pltpu.sync_copy(ridx.at[pl.ds(rbase, _RB)], idx_v)

    def desc(j, buf, sem):
        return pltpu.make_async_copy(
            rtab.at[idx_v.at[pl.ds(j * _CHUNK, _CHUNK)]], buf, sem)

    nch = _RB // _CHUNK
    desc(0, buf0, sem0).start()

    def rstep(j2, carry):
        c = 2 * j2
        desc(c + 1, buf1, sem1).start()
        desc(c, buf0, sem0).wait()
        pltpu.sync_copy(buf0, r_out.at[pl.ds(rbase + c * _CHUNK, _CHUNK)])

        @pl.when(j2 < nch // 2 - 1)
        def _():
            desc(c + 2, buf0, sem0).start()

        desc(c + 1, buf1, sem1).wait()
        pltpu.sync_copy(buf1, r_out.at[pl.ds(rbase + (c + 1) * _CHUNK, _CHUNK)])
        return carry

    lax.fori_loop(0, nch // 2, rstep, 0)


_sc_gather_rel = functools.partial(
    pl.kernel,
    out_type=jax.ShapeDtypeStruct((2 * _B, _D), jnp.float32),
    mesh=plsc.VectorSubcoreMesh(core_axis_name="c", subcore_axis_name="s"),
    compiler_params=pltpu.CompilerParams(use_tc_tiling_on_sc=False),
    scratch_types=[
        pltpu.VMEM((_RB,), jnp.int32),
        pltpu.VMEM((_CHUNK, _D), jnp.float32),
        pltpu.VMEM((_CHUNK, _D), jnp.float32),
        pltpu.SemaphoreType.DMA,
        pltpu.SemaphoreType.DMA,
    ],
)(_sc_gather_rel_body)


_BLK = 1024
_NB = _B // _BLK


def _score_body(xh, xt, xnh, xnt, ih, it, inh, intt, rp, rn, pos_o, neg_o):
    def sel(x_ref, i_ref):
        x = x_ref[...]
        ph = (i_ref[...] >> _SH) & 1
        return jnp.where(ph[:, None] == 1, x[:, _D:], x[:, :_D])

    fh = sel(xh, ih)
    ft = sel(xt, it)
    fnh = sel(xnh, inh)
    fnt = sel(xnt, intt)
    pos_o[...] = jnp.sum(jnp.abs(fh + rp[...] - ft), axis=1)
    neg_o[...] = jnp.sum(jnp.abs(fnh + rn[...] - fnt), axis=1)


def _x_spec(seg):
    return pl.BlockSpec((_BLK, 2 * _D), lambda i, s=seg: (i + s * _NB, 0))


def _i_spec(seg):
    return pl.BlockSpec((_BLK,), lambda i: (i,))


def _r_spec(seg):
    return pl.BlockSpec((_BLK, _D), lambda i, s=seg: (i + s * _NB, 0))


_score = pl.pallas_call(
    _score_body,
    grid=(_NB,),
    in_specs=[
        _x_spec(0), _x_spec(1), _x_spec(2), _x_spec(3),
        _i_spec(0), _i_spec(1), _i_spec(2), _i_spec(3),
        _r_spec(0), _r_spec(1),
    ],
    out_specs=[
        pl.BlockSpec((_BLK,), lambda i: (i,)),
        pl.BlockSpec((_BLK,), lambda i: (i,)),
    ],
    out_shape=[
        jax.ShapeDtypeStruct((_B,), jnp.float32),
        jax.ShapeDtypeStruct((_B,), jnp.float32),
    ],
)


def kernel(pos_h, pos_r, pos_t, neg_h, neg_r, neg_t, entity_table,
           relation_table, commonsense_table, W_fuse, b_fuse):
    eidx = jnp.concatenate([pos_h, pos_t, neg_h, neg_t])
    ridx = jnp.concatenate([pos_r, neg_r])
    ptab = _prep(entity_table.T, commonsense_table.T,
                 W_fuse[:, :_D].astype(jnp.bfloat16),
                 W_fuse[:, _D:].astype(jnp.bfloat16), b_fuse.reshape(1, _D))
    r_rows = _sc_gather_rel(ridx, relation_table)
    x_rows = _sc_gather_x(eidx, ptab)
    pos, neg = _score(
        x_rows, x_rows, x_rows, x_rows,
        pos_h, pos_t, neg_h, neg_t,
        r_rows, r_rows)
    return (pos, neg)
